# Initial kernel scaffold; baseline (speedup 1.0000x reference)
import jax
import jax.numpy as jnp
from jax.experimental import pallas as pl


def kernel(patches):
    raise NotImplementedError("write your pallas kernel here")


# speedup vs baseline: 1.4596x; 1.4596x over previous
import functools

import numpy as np
import jax
import jax.numpy as jnp
from jax import lax
from jax.experimental import pallas as pl
from jax.experimental.pallas import tpu as pltpu
from jax.experimental.pallas import tpu_sc as plsc

_T, _B, _C = 1024, 64, 192
_RATIO = 0.75
_REMAIN_T = int(_T * (1 - _RATIO))

_NC, _NS = 2, 16
_NW = _NC * _NS
_ROWS = _REMAIN_T * _B
_R_PER_W = _ROWS // _NW
_CHUNK = 128
_NCHUNK = _R_PER_W // _CHUNK


def _host_indexes():
    side = int(_T**0.5)
    mask_t = side * side - _REMAIN_T
    block_side = int(mask_t**0.5)
    rng = np.random.RandomState(0)
    fwd, bwd = [], []
    for _ in range(_B):
        i = rng.randint(0, side - block_side + 1)
        j = rng.randint(0, side - block_side + 1)
        mask = np.zeros((side, side), dtype=np.float32)
        mask[i : i + block_side, j : j + block_side] = 1
        mask = mask.flatten()
        f = np.where(mask == 0)[0]
        b = np.argsort(np.concatenate((f, np.where(mask == 1)[0])))
        fwd.append(f)
        bwd.append(b)
    forward = np.stack(fwd, axis=-1).astype(np.int32)
    backward = np.stack(bwd, axis=-1).astype(np.int32)
    return forward, backward


_FWD_NP, _BWD_NP = _host_indexes()
_FLAT_IDX_NP = (
    _FWD_NP[:_REMAIN_T].astype(np.int64) * _B + np.arange(_B, dtype=np.int64)[None, :]
).astype(np.int32).reshape(_NW, _NCHUNK, _CHUNK)


@functools.cache
def _build_sc_gather():
    @functools.partial(
        pl.kernel,
        out_type=jax.ShapeDtypeStruct((_ROWS, _C), jnp.float32),
        mesh=plsc.VectorSubcoreMesh(
            core_axis_name="c", subcore_axis_name="s", num_cores=_NC, num_subcores=_NS
        ),
        scratch_types=[
            pltpu.VMEM((_NCHUNK, _CHUNK), jnp.int32),
            pltpu.VMEM((_R_PER_W, _C), jnp.float32),
            pltpu.SemaphoreType.DMA,
        ],
        compiler_params=pltpu.CompilerParams(use_tc_tiling_on_sc=False),
    )
    def _sc_gather(table_hbm, idx_hbm, out_hbm, idx_v, rows_v, sem):
        wid = lax.axis_index("s") * _NC + lax.axis_index("c")
        pltpu.sync_copy(idx_hbm.at[wid], idx_v)
        copies = [
            pltpu.async_copy(
                table_hbm.at[idx_v.at[c]],
                rows_v.at[pl.ds(c * _CHUNK, _CHUNK)],
                sem,
            )
            for c in range(_NCHUNK)
        ]
        for cp in copies:
            cp.wait()
        pltpu.sync_copy(rows_v, out_hbm.at[pl.ds(wid * _R_PER_W, _R_PER_W)])

    return _sc_gather


def kernel(patches):
    table = patches.reshape(_T * _B, _C)
    idx = jnp.asarray(_FLAT_IDX_NP)
    out = _build_sc_gather()(table, idx)
    masked = out.reshape(_REMAIN_T, _B, _C)
    return masked, jnp.asarray(_FWD_NP), jnp.asarray(_BWD_NP)


# speedup vs baseline: 1.7626x; 1.2076x over previous
import functools

import numpy as np
import jax
import jax.numpy as jnp
from jax import lax
from jax.experimental import pallas as pl
from jax.experimental.pallas import tpu as pltpu
from jax.experimental.pallas import tpu_sc as plsc

_T, _B, _C = 1024, 64, 192
_RATIO = 0.75
_REMAIN_T = int(_T * (1 - _RATIO))
_CSPLIT = 128
_CTAIL = _C - _CSPLIT

_NC, _NS = 2, 16
_NW = _NC * _NS
_ROWS = _REMAIN_T * _B
_R_PER_W = _ROWS // _NW
_CHUNK = 128
_NCHUNK = _R_PER_W // _CHUNK


def _host_indexes():
    side = int(_T**0.5)
    mask_t = side * side - _REMAIN_T
    block_side = int(mask_t**0.5)
    rng = np.random.RandomState(0)
    fwd, bwd = [], []
    for _ in range(_B):
        i = rng.randint(0, side - block_side + 1)
        j = rng.randint(0, side - block_side + 1)
        mask = np.zeros((side, side), dtype=np.float32)
        mask[i : i + block_side, j : j + block_side] = 1
        mask = mask.flatten()
        f = np.where(mask == 0)[0]
        b = np.argsort(np.concatenate((f, np.where(mask == 1)[0])))
        fwd.append(f)
        bwd.append(b)
    forward = np.stack(fwd, axis=-1).astype(np.int32)
    backward = np.stack(bwd, axis=-1).astype(np.int32)
    return forward, backward


_FWD_NP, _BWD_NP = _host_indexes()
_FLAT_IDX_NP = (
    _FWD_NP[:_REMAIN_T].astype(np.int64) * _B + np.arange(_B, dtype=np.int64)[None, :]
).astype(np.int32).reshape(_NW, _NCHUNK, _CHUNK)
_FWD_T_NP = np.ascontiguousarray(_FWD_NP[:_REMAIN_T].T)[:, None, :].astype(np.int32)


@functools.cache
def _build_sc_gather():
    @functools.partial(
        pl.kernel,
        out_type=jax.ShapeDtypeStruct((_ROWS, _C), jnp.float32),
        mesh=plsc.VectorSubcoreMesh(
            core_axis_name="c", subcore_axis_name="s", num_cores=_NC, num_subcores=_NS
        ),
        scratch_types=[
            pltpu.VMEM((_NCHUNK, _CHUNK), jnp.int32),
            pltpu.VMEM((2, _CHUNK, _CSPLIT), jnp.float32),
            pltpu.SemaphoreType.DMA,
            pltpu.SemaphoreType.DMA,
        ],
    )
    def _sc_gather(table_hbm, idx_hbm, out_hbm, idx_v, buf, sem_g, sem_w):
        wid = lax.axis_index("s") * _NC + lax.axis_index("c")
        pltpu.sync_copy(idx_hbm.at[wid], idx_v)

        def fire_gather(c):
            return pltpu.async_copy(
                table_hbm.at[idx_v.at[c], pl.ds(0, _CSPLIT)], buf.at[c % 2], sem_g
            )

        def fire_write(c):
            rows = pl.ds(wid * _R_PER_W + c * _CHUNK, _CHUNK)
            return pltpu.async_copy(
                buf.at[c % 2], out_hbm.at[rows, pl.ds(0, _CSPLIT)], sem_w
            )

        g = [None] * _NCHUNK
        w = [None] * _NCHUNK
        g[0] = fire_gather(0)
        for c in range(_NCHUNK):
            if c + 1 < _NCHUNK:
                if c + 1 >= 2:
                    w[c - 1].wait()
                g[c + 1] = fire_gather(c + 1)
            g[c].wait()
            w[c] = fire_write(c)
        w[_NCHUNK - 2].wait()
        w[_NCHUNK - 1].wait()

    return _sc_gather


_BGRP = 8
_NSTEP = _B // _BGRP


def _tc_tail_body(
    out_alias_ref, patches_ref, fwd_ref, out_ref, tail_v, fwd_v, res_v, sem_in, sem_fw, sem_out
):
    del out_alias_ref
    g = pl.program_id(0)

    def in_copy(step, slot):
        cols = pl.ds(_BGRP * step, _BGRP)
        return (
            pltpu.make_async_copy(
                patches_ref.at[:, cols, pl.ds(_CSPLIT, _CTAIL)],
                tail_v.at[slot],
                sem_in.at[slot],
            ),
            pltpu.make_async_copy(fwd_ref.at[cols], fwd_v.at[slot], sem_fw.at[slot]),
        )

    def out_copy(step, slot):
        cols = pl.ds(_BGRP * step, _BGRP)
        return pltpu.make_async_copy(
            res_v.at[slot],
            out_ref.at[:, cols, pl.ds(_CSPLIT, _CTAIL)],
            sem_out.at[slot],
        )

    slot = lax.rem(g, 2)
    nslot = lax.rem(g + 1, 2)

    @pl.when(g == 0)
    def _():
        for cp in in_copy(0, 0):
            cp.start()

    @pl.when(g + 1 < _NSTEP)
    def _():
        for cp in in_copy(g + 1, nslot):
            cp.start()

    for cp in in_copy(g, slot):
        cp.wait()

    @pl.when(g >= 2)
    def _():
        out_copy(g - 2, slot).wait()

    token = jax.lax.broadcasted_iota(jnp.int32, (_REMAIN_T, _T), 1)
    for j in range(_BGRP):
        fw = fwd_v[slot, j, :]
        onehot = (token == fw[:, None]).astype(jnp.float32)
        res_v[slot, :, j, :] = jnp.dot(
            onehot, tail_v[slot, :, j, :], preferred_element_type=jnp.float32
        )

    out_copy(g, slot).start()

    @pl.when(g == _NSTEP - 1)
    def _():
        out_copy(_NSTEP - 2, nslot).wait()
        out_copy(_NSTEP - 1, slot).wait()


@functools.cache
def _build_tc_tail():
    return pl.pallas_call(
        _tc_tail_body,
        out_shape=jax.ShapeDtypeStruct((_REMAIN_T, _B, _C), jnp.float32),
        grid=(_NSTEP,),
        in_specs=[
            pl.BlockSpec(memory_space=pl.ANY),
            pl.BlockSpec(memory_space=pl.ANY),
            pl.BlockSpec(memory_space=pl.ANY),
        ],
        out_specs=pl.BlockSpec(memory_space=pl.ANY),
        scratch_shapes=[
            pltpu.VMEM((2, _T, _BGRP, _CTAIL), jnp.float32),
            pltpu.VMEM((2, _BGRP, _REMAIN_T), jnp.int32),
            pltpu.VMEM((2, _REMAIN_T, _BGRP, _CTAIL), jnp.float32),
            pltpu.SemaphoreType.DMA((2,)),
            pltpu.SemaphoreType.DMA((2,)),
            pltpu.SemaphoreType.DMA((2,)),
        ],
        input_output_aliases={0: 0},
        compiler_params=pltpu.CompilerParams(
            dimension_semantics=("arbitrary",),
        ),
    )


def kernel(patches):
    table = patches.reshape(_T * _B, _C)
    idx = jnp.asarray(_FLAT_IDX_NP)
    head = _build_sc_gather()(table, idx).reshape(_REMAIN_T, _B, _C)
    fwd_t = jnp.asarray(np.ascontiguousarray(_FWD_NP[:_REMAIN_T].T))
    masked = _build_tc_tail()(head, patches, fwd_t)
    return masked, jnp.asarray(_FWD_NP), jnp.asarray(_BWD_NP)


# speedup vs baseline: 3.9047x; 2.2153x over previous
import functools

import numpy as np
import jax
import jax.numpy as jnp
from jax import lax
from jax.experimental import pallas as pl
from jax.experimental.pallas import tpu as pltpu
from jax.experimental.pallas import tpu_sc as plsc

_T, _B, _C = 1024, 64, 192
_RATIO = 0.75
_REMAIN_T = int(_T * (1 - _RATIO))

_NC, _NS = 2, 16
_NW = _NC * _NS
_CBLK = 32
_NCB = _C // _CBLK
_NTASK = _B * _NCB
_TPW = _NTASK // _NW
_L = 16


def _host_indexes():
    side = int(_T**0.5)
    mask_t = side * side - _REMAIN_T
    block_side = int(mask_t**0.5)
    rng = np.random.RandomState(0)
    fwd, bwd = [], []
    for _ in range(_B):
        i = rng.randint(0, side - block_side + 1)
        j = rng.randint(0, side - block_side + 1)
        mask = np.zeros((side, side), dtype=np.float32)
        mask[i : i + block_side, j : j + block_side] = 1
        mask = mask.flatten()
        f = np.where(mask == 0)[0]
        b = np.argsort(np.concatenate((f, np.where(mask == 1)[0])))
        fwd.append(f)
        bwd.append(b)
    forward = np.stack(fwd, axis=-1).astype(np.int32)
    backward = np.stack(bwd, axis=-1).astype(np.int32)
    return forward, backward


_FWD_NP, _BWD_NP = _host_indexes()
_IDXT_NP = np.ascontiguousarray(_FWD_NP[:_REMAIN_T].T).astype(np.int32)


@functools.cache
def _build_sc_gather():
    @functools.partial(
        pl.kernel,
        out_type=jax.ShapeDtypeStruct((_B, _C, _REMAIN_T), jnp.float32),
        mesh=plsc.VectorSubcoreMesh(
            core_axis_name="c", subcore_axis_name="s", num_cores=_NC, num_subcores=_NS
        ),
        scratch_types=[
            pltpu.VMEM((2, _CBLK, _T), jnp.float32),
            pltpu.VMEM((2, _CBLK, _REMAIN_T), jnp.float32),
            pltpu.VMEM((2, _REMAIN_T), jnp.int32),
            pltpu.SemaphoreType.DMA,
            pltpu.SemaphoreType.DMA,
        ],
        compiler_params=pltpu.CompilerParams(needs_layout_passes=False),
    )
    def _sc_gather(pt_hbm, idx_hbm, out_hbm, inbuf, outbuf, idx_v, sem_g, sem_w):
        wid = lax.axis_index("s") * _NC + lax.axis_index("c")
        task0 = wid * _TPW

        def stage(k, s):
            tk = task0 + k
            b, cb = tk // _NCB, tk % _NCB
            return (
                pltpu.async_copy(
                    pt_hbm.at[pl.ds(b, 1), pl.ds(cb * _CBLK, _CBLK)],
                    inbuf.at[pl.ds(s, 1)],
                    sem_g,
                ),
                pltpu.async_copy(
                    idx_hbm.at[pl.ds(b, 1)], idx_v.at[pl.ds(s, 1)], sem_g
                ),
            )

        def write(k, s):
            tk = task0 + k
            b, cb = tk // _NCB, tk % _NCB
            return pltpu.async_copy(
                outbuf.at[pl.ds(s, 1)],
                out_hbm.at[pl.ds(b, 1), pl.ds(cb * _CBLK, _CBLK)],
                sem_w,
            )

        def compute(s):
            slot = jnp.full((_L,), s, jnp.int32)

            def jbody(j, _):
                col = idx_v[s, pl.ds(j * _L, _L)]
                for c in range(_CBLK):
                    row = jnp.full((_L,), c, jnp.int32)
                    outbuf[s, c, pl.ds(j * _L, _L)] = plsc.load_gather(
                        inbuf, [slot, row, col]
                    )
                return _

            lax.fori_loop(0, _REMAIN_T // _L, jbody, 0)

        g = [None] * _TPW
        w = [None] * _TPW
        g[0] = stage(0, 0)
        for k in range(_TPW):
            s = k % 2
            if k >= 2:
                w[k - 2].wait()
            if k + 1 < _TPW:
                g[k + 1] = stage(k + 1, 1 - s)
            for cp in g[k]:
                cp.wait()
            compute(s)
            w[k] = write(k, s)
        w[_TPW - 2].wait()
        w[_TPW - 1].wait()

    return _sc_gather


def kernel(patches):
    pt = lax.transpose(patches, (1, 2, 0))
    out_pt = _build_sc_gather()(pt, jnp.asarray(_IDXT_NP))
    masked = lax.transpose(out_pt, (2, 0, 1))
    fwd = lax.transpose(jnp.asarray(np.ascontiguousarray(_FWD_NP.T)), (1, 0))
    bwd = lax.transpose(jnp.asarray(np.ascontiguousarray(_BWD_NP.T)), (1, 0))
    return masked, fwd, bwd
